# baseline (speedup 1.0000x reference)
import functools

import jax
import jax.numpy as jnp
import numpy as np
from jax import lax
from jax.experimental import pallas as pl
from jax.experimental.pallas import tpu as pltpu
from jax.experimental.pallas import tpu_sc as plsc

_N = 10000
_D = 128
_E = 320000
_NC = 2
_NS = 16
_NW = _NC * _NS
_BLK = 128
_NBLKP = 2560
_EPAD = _NBLKP * _BLK - _E
_BASE = _NBLKP // _NW
_NRING = 2
_DRING = 4
_SB = 8
_NSB = _BASE // _SB
_RPW = 632
_RPW_LAST = _N - (_NS - 1) * _RPW

def _block_perm():
  perm, r, p = [], 0, _E // _BLK
  nreal_total = _E // _BLK
  for w in range(_NW):
    nreal = 79 if w < nreal_total - 78 * _NW else 78
    perm.extend(range(r, r + nreal))
    r += nreal
    perm.extend(range(p, p + 80 - nreal))
    p += 80 - nreal
  return np.asarray(perm, np.int32)

_BPERM = _block_perm()


def _worker_id():
  return lax.axis_index("s") * _NC + lax.axis_index("c")


def _fill(buf, nrows, value16):
  def frow(i, carry):
    for j in range(_D // 16):
      buf[i, pl.ds(j * 16, 16)] = value16
    return carry

  lax.fori_loop(0, nrows, frow, 0)


def _zero_spmem(acc_sh, zbuf, base, nrows):
  chunk = zbuf.shape[0]
  for t in range(nrows // chunk):
    pltpu.sync_copy(zbuf, acc_sh.at[pl.ds(base + t * chunk, chunk)])
  rem = nrows - (nrows // chunk) * chunk
  if rem:
    pltpu.sync_copy(zbuf.at[pl.ds(0, rem)],
                    acc_sh.at[pl.ds(base + nrows - rem, rem)])


def _per_tile(body_aligned, body_last):
  s = lax.axis_index("s")

  @pl.when(s < _NS - 1)
  def _():
    body_aligned()

  @pl.when(s == _NS - 1)
  def _():
    body_last()


def _make_segsum():
  mesh = plsc.VectorSubcoreMesh(core_axis_name="c", subcore_axis_name="s")

  @functools.partial(
      pl.kernel,
      out_type=jax.ShapeDtypeStruct((_NC, _N, _D), jnp.float32),
      mesh=mesh,
      scratch_types=[
          pltpu.VMEM((_SB, _BLK), jnp.int32),
          pltpu.VMEM((_SB, _BLK), jnp.int32),
          pltpu.VMEM((_NRING, _BLK, _D), jnp.float32),
          pltpu.VMEM_SHARED((_N, _D), jnp.float32),
      ] + [pltpu.SemaphoreType.DMA] * (2 * _NRING),
  )
  def seg(table_h, src_h, dst_h, acc_out, src_v, dst_v, rows_v, acc_sh,
          *sems):
    gsems, ssems = sems[:_NRING], sems[_NRING:]
    c = lax.axis_index("c")
    s = lax.axis_index("s")
    wid = _worker_id()

    _fill(rows_v.at[0], _BLK, jnp.zeros((16,), jnp.float32))
    base = s * _RPW
    _per_tile(lambda: _zero_spmem(acc_sh, rows_v.at[0], base, _RPW),
              lambda: _zero_spmem(acc_sh, rows_v.at[0], base, _RPW_LAST))
    plsc.subcore_barrier()

    def gfire(k):
      return pltpu.async_copy(table_h.at[src_v.at[k]],
                              rows_v.at[k % _NRING], gsems[k % _NRING])

    def sfire(k):
      return pltpu.async_copy(rows_v.at[k % _NRING],
                              acc_sh.at[dst_v.at[k]], ssems[k % _NRING],
                              add=True)

    def supergroup(g, carry):
      b0 = pl.multiple_of((wid * _NSB + g) * _SB, 8)
      pltpu.sync_copy(src_h.at[pl.ds(b0, _SB)], src_v)
      pltpu.sync_copy(dst_h.at[pl.ds(b0, _SB)], dst_v)
      gcps = {}
      for k in range(_SB):
        if k >= _NRING:
          gcps.pop(k - _NRING).wait()
          sfire(k - _NRING).wait()
        gcps[k] = gfire(k)
      for k in range(_SB - _NRING, _SB):
        gcps.pop(k).wait()
        sfire(k).wait()
      return carry

    lax.fori_loop(0, _NSB, supergroup, 0)

    plsc.subcore_barrier()

    def wout(nrows):
      pltpu.sync_copy(acc_sh.at[pl.ds(base, nrows)],
                      acc_out.at[c, pl.ds(base, nrows)])

    _per_tile(lambda: wout(_RPW), lambda: wout(_RPW_LAST))

  return seg


def _make_deg():
  mesh = plsc.VectorSubcoreMesh(core_axis_name="c", subcore_axis_name="s")

  @functools.partial(
      pl.kernel,
      out_type=jax.ShapeDtypeStruct((_NC, _N, _D), jnp.float32),
      mesh=mesh,
      scratch_types=[
          pltpu.VMEM((_SB, _BLK), jnp.int32),
          pltpu.VMEM((_BLK, _D), jnp.float32),
          pltpu.VMEM((64, _D), jnp.float32),
          pltpu.VMEM_SHARED((_N, _D), jnp.float32),
      ] + [pltpu.SemaphoreType.DMA] * _DRING,
  )
  def deg(dst_h, deg_out, dst_v, ones_v, zros_v, acc_sh, *sems):
    c = lax.axis_index("c")
    s = lax.axis_index("s")
    wid = _worker_id()

    _fill(ones_v, _BLK, jnp.ones((16,), jnp.float32))
    _fill(zros_v, 64, jnp.zeros((16,), jnp.float32))
    base = s * _RPW
    _per_tile(lambda: _zero_spmem(acc_sh, zros_v, base, _RPW),
              lambda: _zero_spmem(acc_sh, zros_v, base, _RPW_LAST))
    plsc.subcore_barrier()

    def fire(k):
      return pltpu.async_copy(ones_v, acc_sh.at[dst_v.at[k]],
                              sems[k % _DRING], add=True)

    def supergroup(g, carry):
      b0 = pl.multiple_of((wid * _NSB + g) * _SB, 8)
      pltpu.sync_copy(dst_h.at[pl.ds(b0, _SB)], dst_v)
      cps = {}
      for k in range(_SB):
        if k >= _DRING:
          cps.pop(k - _DRING).wait()
        cps[k] = fire(k)
      for k in range(max(_SB - _DRING, 0), _SB):
        cps.pop(k).wait()
      return carry

    lax.fori_loop(0, _NSB, supergroup, 0)

    plsc.subcore_barrier()

    def wout(nrows):
      pltpu.sync_copy(acc_sh.at[pl.ds(base, nrows)],
                      deg_out.at[c, pl.ds(base, nrows)])

    _per_tile(lambda: wout(_RPW), lambda: wout(_RPW_LAST))

  return deg


@functools.lru_cache(maxsize=None)
def _get_segsum():
  return _make_segsum()


@functools.lru_cache(maxsize=None)
def _get_deg():
  return _make_deg()


_R = 1000


def _matmul(a, wT):
  grid = (_N // _R,)
  return pl.pallas_call(
      lambda aa, ww, oo: oo.__setitem__(
          ..., jnp.dot(aa[...], ww[...],
                       preferred_element_type=jnp.float32)),
      grid=grid,
      in_specs=[
          pl.BlockSpec((_R, _D), lambda i: (i, 0)),
          pl.BlockSpec((_D, _D), lambda i: (0, 0)),
      ],
      out_specs=pl.BlockSpec((_R, _D), lambda i: (i, 0)),
      out_shape=jax.ShapeDtypeStruct((_N, _D), jnp.float32),
  )(a, wT)


def _dense1_body(accp, degp, xr, wl, b, g, be, h_out, inv_out):
  ssum = accp[0] + accp[1]
  dg = (degp[0] + degp[1])[:, 0:1]
  rid = (lax.broadcasted_iota(jnp.int32, (_R, 1), 0)
         + pl.program_id(0) * _R)
  dg = dg - jnp.where(rid < _EPAD, 1.0, 0.0)
  invd = 1.0 / jnp.maximum(dg, 1.0)
  mean = ssum * invd
  h = (jnp.dot(mean, wl[...], preferred_element_type=jnp.float32)
       + xr[...] + b[...])
  mu = jnp.mean(h, axis=1, keepdims=True)
  var = jnp.mean((h - mu) ** 2, axis=1, keepdims=True)
  h = (h - mu) * lax.rsqrt(var + 1e-5) * g[...] + be[...]
  h_out[...] = jnp.maximum(h, 0.0)
  inv_out[...] = jnp.broadcast_to(invd, (_R, _D))


def _dense1(acc1, deg1, xr, wlT, b, g, be):
  grid = (_N // _R,)
  return pl.pallas_call(
      lambda a, d, xx, wl, bb, gg, bbe, ho, io: _dense1_body(
          a[...], d[...], xx, wl, bb, gg, bbe, ho, io),
      grid=grid,
      in_specs=[
          pl.BlockSpec((_NC, _R, _D), lambda i: (0, i, 0)),
          pl.BlockSpec((_NC, _R, _D), lambda i: (0, i, 0)),
          pl.BlockSpec((_R, _D), lambda i: (i, 0)),
          pl.BlockSpec((_D, _D), lambda i: (0, 0)),
          pl.BlockSpec((1, _D), lambda i: (0, 0)),
          pl.BlockSpec((1, _D), lambda i: (0, 0)),
          pl.BlockSpec((1, _D), lambda i: (0, 0)),
      ],
      out_specs=[
          pl.BlockSpec((_R, _D), lambda i: (i, 0)),
          pl.BlockSpec((_R, _D), lambda i: (i, 0)),
      ],
      out_shape=[
          jax.ShapeDtypeStruct((_N, _D), jnp.float32),
          jax.ShapeDtypeStruct((_N, _D), jnp.float32),
      ],
  )(acc1, deg1, xr, wlT, b, g, be)


def _dense2_body(accp, invd, hr, wl, b, out):
  mean = (accp[0] + accp[1]) * invd[...]
  o = (jnp.dot(mean, wl[...], preferred_element_type=jnp.float32)
       + hr[...] + b[...])
  nrm = jnp.sqrt(jnp.sum(o * o, axis=1, keepdims=True))
  out[...] = o / jnp.maximum(nrm, 1e-12)


def _dense2(acc2, invd, hr, wlT, b):
  grid = (_N // _R,)
  return pl.pallas_call(
      lambda a, iv, hh, wl, bb, oo: _dense2_body(
          a[...], iv, hh, wl, bb, oo),
      grid=grid,
      in_specs=[
          pl.BlockSpec((_NC, _R, _D), lambda i: (0, i, 0)),
          pl.BlockSpec((_R, _D), lambda i: (i, 0)),
          pl.BlockSpec((_R, _D), lambda i: (i, 0)),
          pl.BlockSpec((_D, _D), lambda i: (0, 0)),
          pl.BlockSpec((1, _D), lambda i: (0, 0)),
      ],
      out_specs=pl.BlockSpec((_R, _D), lambda i: (i, 0)),
      out_shape=jax.ShapeDtypeStruct((_N, _D), jnp.float32),
  )(acc2, invd, hr, wlT, b)


def kernel(x, edge_index, W1l, b1l, W1r, g1, be1, W2l, b2l, W2r):
  padi = jnp.arange(_EPAD, dtype=jnp.int32)
  src2d = jnp.concatenate(
      [edge_index[0], _N + (padi % 8)]).reshape(_NBLKP, _BLK)[_BPERM]
  dst2d = jnp.concatenate(
      [edge_index[1], padi]).reshape(_NBLKP, _BLK)[_BPERM]
  zrow = jnp.zeros((8, _D), jnp.float32)
  xp = jnp.concatenate([x, zrow])
  acc1 = _get_segsum()(xp, src2d, dst2d)
  deg1 = _get_deg()(dst2d)
  xr = _matmul(x, W1r.T)
  h, invd = _dense1(acc1, deg1, xr, W1l.T,
                    b1l.reshape(1, _D), g1.reshape(1, _D),
                    be1.reshape(1, _D))
  acc2 = _get_segsum()(jnp.concatenate([h, zrow]), src2d, dst2d)
  hr = _matmul(h, W2r.T)
  return _dense2(acc2, invd, hr, W2l.T, b2l.reshape(1, _D))


# speedup vs baseline: 1.1445x; 1.1445x over previous
import functools

import jax
import jax.numpy as jnp
from jax import lax
from jax.experimental import pallas as pl
from jax.experimental.pallas import tpu as pltpu
from jax.experimental.pallas import tpu_sc as plsc

_N = 10000
_D = 128
_E = 320000
_NC = 2
_NS = 16
_NW = _NC * _NS
_BLK = 128
_NBLKP = 2560
_EPAD = _NBLKP * _BLK - _E
_BASE = _NBLKP // _NW
_NRING = 2
_DRING = 4
_SB = 8
_NSB = _BASE // _SB
_RPW = 632
_RPW_LAST = _N - (_NS - 1) * _RPW


def _worker_id():
  return lax.axis_index("s") * _NC + lax.axis_index("c")


def _fill(buf, nrows, value16):
  def frow(i, carry):
    for j in range(_D // 16):
      buf[i, pl.ds(j * 16, 16)] = value16
    return carry

  lax.fori_loop(0, nrows, frow, 0)


def _zero_spmem(acc_sh, zbuf, base, nrows):
  chunk = zbuf.shape[0]
  for t in range(nrows // chunk):
    pltpu.sync_copy(zbuf, acc_sh.at[pl.ds(base + t * chunk, chunk)])
  rem = nrows - (nrows // chunk) * chunk
  if rem:
    pltpu.sync_copy(zbuf.at[pl.ds(0, rem)],
                    acc_sh.at[pl.ds(base + nrows - rem, rem)])


def _per_tile(body_aligned, body_last):
  s = lax.axis_index("s")

  @pl.when(s < _NS - 1)
  def _():
    body_aligned()

  @pl.when(s == _NS - 1)
  def _():
    body_last()


def _make_segsum():
  mesh = plsc.VectorSubcoreMesh(core_axis_name="c", subcore_axis_name="s")

  @functools.partial(
      pl.kernel,
      out_type=jax.ShapeDtypeStruct((_NC, _N, _D), jnp.float32),
      mesh=mesh,
      scratch_types=[
          pltpu.VMEM((2, _SB, _BLK), jnp.int32),
          pltpu.VMEM((2, _SB, _BLK), jnp.int32),
          pltpu.VMEM((_NRING, _BLK, _D), jnp.float32),
          pltpu.VMEM_SHARED((_N, _D), jnp.float32),
      ] + [pltpu.SemaphoreType.DMA] * (2 * _NRING + 2),
  )
  def seg(table_h, src_h, dst_h, acc_out, src_v, dst_v, rows_v, acc_sh,
          *sems):
    gsems, ssems = sems[:_NRING], sems[_NRING:2 * _NRING]
    isems = sems[2 * _NRING:]
    c = lax.axis_index("c")
    s = lax.axis_index("s")
    wid = _worker_id()

    _fill(rows_v.at[0], _BLK, jnp.zeros((16,), jnp.float32))
    base = s * _RPW
    _per_tile(lambda: _zero_spmem(acc_sh, rows_v.at[0], base, _RPW),
              lambda: _zero_spmem(acc_sh, rows_v.at[0], base, _RPW_LAST))
    plsc.subcore_barrier()

    def gfire(pb, k):
      return pltpu.async_copy(table_h.at[src_v.at[pb, k]],
                              rows_v.at[k % _NRING], gsems[k % _NRING])

    def sfire(pb, k):
      return pltpu.async_copy(rows_v.at[k % _NRING],
                              acc_sh.at[dst_v.at[pb, k]],
                              ssems[k % _NRING], add=True)

    def iload(g, pb):
      b0 = pl.multiple_of((wid * _NSB + g) * _SB, 8)
      c0 = pltpu.async_copy(src_h.at[pl.ds(b0, _SB)], src_v.at[pb],
                            isems[0])
      c1 = pltpu.async_copy(dst_h.at[pl.ds(b0, _SB)], dst_v.at[pb],
                            isems[1])
      return c0, c1

    def process(pb):
      gcps = {}
      for k in range(_SB):
        if k >= _NRING:
          gcps.pop(k - _NRING).wait()
          sfire(pb, k - _NRING).wait()
        gcps[k] = gfire(pb, k)
      for k in range(_SB - _NRING, _SB):
        gcps.pop(k).wait()
        sfire(pb, k).wait()

    for cp in iload(0, 0):
      cp.wait()

    def pair(j, carry):
      g = j * 2
      n0, n1 = iload(g + 1, 1)
      process(0)
      n0.wait(), n1.wait()

      @pl.when(j < _NSB // 2 - 1)
      def _():
        m0, m1 = iload(g + 2, 0)
        process(1)
        m0.wait(), m1.wait()

      @pl.when(j == _NSB // 2 - 1)
      def _():
        process(1)

      return carry

    lax.fori_loop(0, _NSB // 2, pair, 0)

    plsc.subcore_barrier()

    def wout(nrows):
      pltpu.sync_copy(acc_sh.at[pl.ds(base, nrows)],
                      acc_out.at[c, pl.ds(base, nrows)])

    _per_tile(lambda: wout(_RPW), lambda: wout(_RPW_LAST))

  return seg


def _make_deg():
  mesh = plsc.VectorSubcoreMesh(core_axis_name="c", subcore_axis_name="s")

  @functools.partial(
      pl.kernel,
      out_type=jax.ShapeDtypeStruct((_NC, _N, _D), jnp.float32),
      mesh=mesh,
      scratch_types=[
          pltpu.VMEM((_SB, _BLK), jnp.int32),
          pltpu.VMEM((_BLK, _D), jnp.float32),
          pltpu.VMEM((64, _D), jnp.float32),
          pltpu.VMEM_SHARED((_N, _D), jnp.float32),
      ] + [pltpu.SemaphoreType.DMA] * _DRING,
  )
  def deg(dst_h, deg_out, dst_v, ones_v, zros_v, acc_sh, *sems):
    c = lax.axis_index("c")
    s = lax.axis_index("s")
    wid = _worker_id()

    _fill(ones_v, _BLK, jnp.ones((16,), jnp.float32))
    _fill(zros_v, 64, jnp.zeros((16,), jnp.float32))
    base = s * _RPW
    _per_tile(lambda: _zero_spmem(acc_sh, zros_v, base, _RPW),
              lambda: _zero_spmem(acc_sh, zros_v, base, _RPW_LAST))
    plsc.subcore_barrier()

    def fire(k):
      return pltpu.async_copy(ones_v, acc_sh.at[dst_v.at[k]],
                              sems[k % _DRING], add=True)

    def supergroup(g, carry):
      b0 = pl.multiple_of((wid * _NSB + g) * _SB, 8)
      pltpu.sync_copy(dst_h.at[pl.ds(b0, _SB)], dst_v)
      cps = {}
      for k in range(_SB):
        if k >= _DRING:
          cps.pop(k - _DRING).wait()
        cps[k] = fire(k)
      for k in range(max(_SB - _DRING, 0), _SB):
        cps.pop(k).wait()
      return carry

    lax.fori_loop(0, _NSB, supergroup, 0)

    plsc.subcore_barrier()

    def wout(nrows):
      pltpu.sync_copy(acc_sh.at[pl.ds(base, nrows)],
                      deg_out.at[c, pl.ds(base, nrows)])

    _per_tile(lambda: wout(_RPW), lambda: wout(_RPW_LAST))

  return deg


@functools.lru_cache(maxsize=None)
def _get_segsum():
  return _make_segsum()


@functools.lru_cache(maxsize=None)
def _get_deg():
  return _make_deg()


_R = 1000


def _matmul(a, wT):
  grid = (_N // _R,)
  return pl.pallas_call(
      lambda aa, ww, oo: oo.__setitem__(
          ..., jnp.dot(aa[...], ww[...],
                       preferred_element_type=jnp.float32)),
      grid=grid,
      in_specs=[
          pl.BlockSpec((_R, _D), lambda i: (i, 0)),
          pl.BlockSpec((_D, _D), lambda i: (0, 0)),
      ],
      out_specs=pl.BlockSpec((_R, _D), lambda i: (i, 0)),
      out_shape=jax.ShapeDtypeStruct((_N, _D), jnp.float32),
  )(a, wT)


def _dense1_body(accp, degp, xr, wl, b, g, be, h_out, inv_out):
  ssum = accp[0] + accp[1]
  dg = (degp[0] + degp[1])[:, 0:1]
  rid = (lax.broadcasted_iota(jnp.int32, (_R, 1), 0)
         + pl.program_id(0) * _R)
  dg = dg - jnp.where(rid < _EPAD, 1.0, 0.0)
  invd = 1.0 / jnp.maximum(dg, 1.0)
  mean = ssum * invd
  h = (jnp.dot(mean, wl[...], preferred_element_type=jnp.float32)
       + xr[...] + b[...])
  mu = jnp.mean(h, axis=1, keepdims=True)
  var = jnp.mean((h - mu) ** 2, axis=1, keepdims=True)
  h = (h - mu) * lax.rsqrt(var + 1e-5) * g[...] + be[...]
  h_out[...] = jnp.maximum(h, 0.0)
  inv_out[...] = jnp.broadcast_to(invd, (_R, _D))


def _dense1(acc1, deg1, xr, wlT, b, g, be):
  grid = (_N // _R,)
  return pl.pallas_call(
      lambda a, d, xx, wl, bb, gg, bbe, ho, io: _dense1_body(
          a[...], d[...], xx, wl, bb, gg, bbe, ho, io),
      grid=grid,
      in_specs=[
          pl.BlockSpec((_NC, _R, _D), lambda i: (0, i, 0)),
          pl.BlockSpec((_NC, _R, _D), lambda i: (0, i, 0)),
          pl.BlockSpec((_R, _D), lambda i: (i, 0)),
          pl.BlockSpec((_D, _D), lambda i: (0, 0)),
          pl.BlockSpec((1, _D), lambda i: (0, 0)),
          pl.BlockSpec((1, _D), lambda i: (0, 0)),
          pl.BlockSpec((1, _D), lambda i: (0, 0)),
      ],
      out_specs=[
          pl.BlockSpec((_R, _D), lambda i: (i, 0)),
          pl.BlockSpec((_R, _D), lambda i: (i, 0)),
      ],
      out_shape=[
          jax.ShapeDtypeStruct((_N, _D), jnp.float32),
          jax.ShapeDtypeStruct((_N, _D), jnp.float32),
      ],
  )(acc1, deg1, xr, wlT, b, g, be)


def _dense2_body(accp, invd, hr, wl, b, out):
  mean = (accp[0] + accp[1]) * invd[...]
  o = (jnp.dot(mean, wl[...], preferred_element_type=jnp.float32)
       + hr[...] + b[...])
  nrm = jnp.sqrt(jnp.sum(o * o, axis=1, keepdims=True))
  out[...] = o / jnp.maximum(nrm, 1e-12)


def _dense2(acc2, invd, hr, wlT, b):
  grid = (_N // _R,)
  return pl.pallas_call(
      lambda a, iv, hh, wl, bb, oo: _dense2_body(
          a[...], iv, hh, wl, bb, oo),
      grid=grid,
      in_specs=[
          pl.BlockSpec((_NC, _R, _D), lambda i: (0, i, 0)),
          pl.BlockSpec((_R, _D), lambda i: (i, 0)),
          pl.BlockSpec((_R, _D), lambda i: (i, 0)),
          pl.BlockSpec((_D, _D), lambda i: (0, 0)),
          pl.BlockSpec((1, _D), lambda i: (0, 0)),
      ],
      out_specs=pl.BlockSpec((_R, _D), lambda i: (i, 0)),
      out_shape=jax.ShapeDtypeStruct((_N, _D), jnp.float32),
  )(acc2, invd, hr, wlT, b)


def kernel(x, edge_index, W1l, b1l, W1r, g1, be1, W2l, b2l, W2r):
  padi = jnp.arange(_EPAD, dtype=jnp.int32)
  src2d = jnp.concatenate(
      [edge_index[0], _N + (padi % 8)]).reshape(_NBLKP, _BLK)
  dst2d = jnp.concatenate([edge_index[1], padi]).reshape(_NBLKP, _BLK)
  zrow = jnp.zeros((8, _D), jnp.float32)
  xp = jnp.concatenate([x, zrow])
  acc1 = _get_segsum()(xp, src2d, dst2d)
  deg1 = _get_deg()(dst2d)
  xr = _matmul(x, W1r.T)
  h, invd = _dense1(acc1, deg1, xr, W1l.T,
                    b1l.reshape(1, _D), g1.reshape(1, _D),
                    be1.reshape(1, _D))
  acc2 = _get_segsum()(jnp.concatenate([h, zrow]), src2d, dst2d)
  hr = _matmul(h, W2r.T)
  return _dense2(acc2, invd, hr, W2l.T, b2l.reshape(1, _D))


# speedup vs baseline: 1.1574x; 1.0113x over previous
import functools

import jax
import jax.numpy as jnp
from jax import lax
from jax.experimental import pallas as pl
from jax.experimental.pallas import tpu as pltpu
from jax.experimental.pallas import tpu_sc as plsc

_N = 10000
_D = 128
_E = 320000
_NC = 2
_NS = 16
_NW = _NC * _NS
_BLK = 128
_NBLKP = 2560
_EPAD = _NBLKP * _BLK - _E
_BASE = _NBLKP // _NW
_NRING = 2
_DRING = 4
_SB = 8
_NSB = _BASE // _SB
_RPW = 632
_RPW_LAST = _N - (_NS - 1) * _RPW


def _worker_id():
  return lax.axis_index("s") * _NC + lax.axis_index("c")


def _fill(buf, nrows, value16):
  def frow(i, carry):
    for j in range(_D // 16):
      buf[i, pl.ds(j * 16, 16)] = value16
    return carry

  lax.fori_loop(0, nrows, frow, 0)


def _zero_spmem(acc_sh, zbuf, base, nrows):
  chunk = zbuf.shape[0]
  for t in range(nrows // chunk):
    pltpu.sync_copy(zbuf, acc_sh.at[pl.ds(base + t * chunk, chunk)])
  rem = nrows - (nrows // chunk) * chunk
  if rem:
    pltpu.sync_copy(zbuf.at[pl.ds(0, rem)],
                    acc_sh.at[pl.ds(base + nrows - rem, rem)])


def _per_tile(body_aligned, body_last):
  s = lax.axis_index("s")

  @pl.when(s < _NS - 1)
  def _():
    body_aligned()

  @pl.when(s == _NS - 1)
  def _():
    body_last()


def _make_segsum():
  mesh = plsc.VectorSubcoreMesh(core_axis_name="c", subcore_axis_name="s")

  @functools.partial(
      pl.kernel,
      out_type=jax.ShapeDtypeStruct((_NC, _N, _D), jnp.float32),
      mesh=mesh,
      scratch_types=[
          pltpu.VMEM((2, _SB, _BLK), jnp.int32),
          pltpu.VMEM((2, _SB, _BLK), jnp.int32),
          pltpu.VMEM((_NRING, _BLK, _D), jnp.float32),
          pltpu.VMEM_SHARED((_N, _D), jnp.float32),
      ] + [pltpu.SemaphoreType.DMA] * (2 * _NRING + 2),
  )
  def seg(table_h, src_h, dst_h, acc_out, src_v, dst_v, rows_v, acc_sh,
          *sems):
    gsems, ssems = sems[:_NRING], sems[_NRING:2 * _NRING]
    isems = sems[2 * _NRING:]
    c = lax.axis_index("c")
    s = lax.axis_index("s")
    wid = _worker_id()

    _fill(rows_v.at[0], _BLK, jnp.zeros((16,), jnp.float32))
    base = s * _RPW
    _per_tile(lambda: _zero_spmem(acc_sh, rows_v.at[0], base, _RPW),
              lambda: _zero_spmem(acc_sh, rows_v.at[0], base, _RPW_LAST))
    plsc.subcore_barrier()

    def gfire(pb, k):
      return pltpu.async_copy(table_h.at[src_v.at[pb, k]],
                              rows_v.at[k % _NRING], gsems[k % _NRING])

    def sfire(pb, k):
      return pltpu.async_copy(rows_v.at[k % _NRING],
                              acc_sh.at[dst_v.at[pb, k]],
                              ssems[k % _NRING], add=True)

    def iload(g, pb):
      b0 = pl.multiple_of((wid * _NSB + g) * _SB, 8)
      c0 = pltpu.async_copy(src_h.at[pl.ds(b0, _SB)], src_v.at[pb],
                            isems[0])
      c1 = pltpu.async_copy(dst_h.at[pl.ds(b0, _SB)], dst_v.at[pb],
                            isems[1])
      return c0, c1

    def process(pb):
      gcps = {}
      for k in range(_SB):
        if k >= _NRING:
          gcps.pop(k - _NRING).wait()
          sfire(pb, k - _NRING).wait()
        gcps[k] = gfire(pb, k)
      for k in range(_SB - _NRING, _SB):
        gcps.pop(k).wait()
        sfire(pb, k).wait()

    for cp in iload(0, 0):
      cp.wait()

    def pair(j, carry):
      g = j * 2
      n0, n1 = iload(g + 1, 1)
      process(0)
      n0.wait(), n1.wait()

      @pl.when(j < _NSB // 2 - 1)
      def _():
        m0, m1 = iload(g + 2, 0)
        process(1)
        m0.wait(), m1.wait()

      @pl.when(j == _NSB // 2 - 1)
      def _():
        process(1)

      return carry

    lax.fori_loop(0, _NSB // 2, pair, 0)

    plsc.subcore_barrier()

    def wout(nrows):
      pltpu.sync_copy(acc_sh.at[pl.ds(base, nrows)],
                      acc_out.at[c, pl.ds(base, nrows)])

    _per_tile(lambda: wout(_RPW), lambda: wout(_RPW_LAST))

  return seg


def _make_deg():
  mesh = plsc.VectorSubcoreMesh(core_axis_name="c", subcore_axis_name="s")

  @functools.partial(
      pl.kernel,
      out_type=jax.ShapeDtypeStruct((_NC, _N, _D), jnp.float32),
      mesh=mesh,
      scratch_types=[
          pltpu.VMEM((2, _SB, _BLK), jnp.int32),
          pltpu.VMEM((_BLK, _D), jnp.float32),
          pltpu.VMEM((64, _D), jnp.float32),
          pltpu.VMEM_SHARED((_N, _D), jnp.float32),
      ] + [pltpu.SemaphoreType.DMA] * (_DRING + 1),
  )
  def deg(dst_h, deg_out, dst_v, ones_v, zros_v, acc_sh, *allsems):
    sems, isem = allsems[:_DRING], allsems[_DRING]
    c = lax.axis_index("c")
    s = lax.axis_index("s")
    wid = _worker_id()

    _fill(ones_v, _BLK, jnp.ones((16,), jnp.float32))
    _fill(zros_v, 64, jnp.zeros((16,), jnp.float32))
    base = s * _RPW
    _per_tile(lambda: _zero_spmem(acc_sh, zros_v, base, _RPW),
              lambda: _zero_spmem(acc_sh, zros_v, base, _RPW_LAST))
    plsc.subcore_barrier()

    def fire(pb, k):
      return pltpu.async_copy(ones_v, acc_sh.at[dst_v.at[pb, k]],
                              sems[k % _DRING], add=True)

    def iload(g, pb):
      b0 = pl.multiple_of((wid * _NSB + g) * _SB, 8)
      return pltpu.async_copy(dst_h.at[pl.ds(b0, _SB)], dst_v.at[pb],
                              isem)

    def process(pb):
      cps = {}
      for k in range(_SB):
        if k >= _DRING:
          cps.pop(k - _DRING).wait()
        cps[k] = fire(pb, k)
      for k in range(max(_SB - _DRING, 0), _SB):
        cps.pop(k).wait()

    iload(0, 0).wait()

    def pair(j, carry):
      g = j * 2
      n0 = iload(g + 1, 1)
      process(0)
      n0.wait()

      @pl.when(j < _NSB // 2 - 1)
      def _():
        m0 = iload(g + 2, 0)
        process(1)
        m0.wait()

      @pl.when(j == _NSB // 2 - 1)
      def _():
        process(1)

      return carry

    lax.fori_loop(0, _NSB // 2, pair, 0)

    plsc.subcore_barrier()

    def wout(nrows):
      pltpu.sync_copy(acc_sh.at[pl.ds(base, nrows)],
                      deg_out.at[c, pl.ds(base, nrows)])

    _per_tile(lambda: wout(_RPW), lambda: wout(_RPW_LAST))

  return deg


@functools.lru_cache(maxsize=None)
def _get_segsum():
  return _make_segsum()


@functools.lru_cache(maxsize=None)
def _get_deg():
  return _make_deg()


_R = 1000


def _matmul(a, wT):
  grid = (_N // _R,)
  return pl.pallas_call(
      lambda aa, ww, oo: oo.__setitem__(
          ..., jnp.dot(aa[...], ww[...],
                       preferred_element_type=jnp.float32)),
      grid=grid,
      in_specs=[
          pl.BlockSpec((_R, _D), lambda i: (i, 0)),
          pl.BlockSpec((_D, _D), lambda i: (0, 0)),
      ],
      out_specs=pl.BlockSpec((_R, _D), lambda i: (i, 0)),
      out_shape=jax.ShapeDtypeStruct((_N, _D), jnp.float32),
  )(a, wT)


def _dense1_body(accp, degp, xr, wl, b, g, be, h_out, inv_out):
  ssum = accp[0] + accp[1]
  dg = (degp[0] + degp[1])[:, 0:1]
  rid = (lax.broadcasted_iota(jnp.int32, (_R, 1), 0)
         + pl.program_id(0) * _R)
  dg = dg - jnp.where(rid < _EPAD, 1.0, 0.0)
  invd = 1.0 / jnp.maximum(dg, 1.0)
  mean = ssum * invd
  h = (jnp.dot(mean, wl[...], preferred_element_type=jnp.float32)
       + xr[...] + b[...])
  mu = jnp.mean(h, axis=1, keepdims=True)
  var = jnp.mean((h - mu) ** 2, axis=1, keepdims=True)
  h = (h - mu) * lax.rsqrt(var + 1e-5) * g[...] + be[...]
  h_out[...] = jnp.maximum(h, 0.0)
  inv_out[...] = jnp.broadcast_to(invd, (_R, _D))


def _dense1(acc1, deg1, xr, wlT, b, g, be):
  grid = (_N // _R,)
  return pl.pallas_call(
      lambda a, d, xx, wl, bb, gg, bbe, ho, io: _dense1_body(
          a[...], d[...], xx, wl, bb, gg, bbe, ho, io),
      grid=grid,
      in_specs=[
          pl.BlockSpec((_NC, _R, _D), lambda i: (0, i, 0)),
          pl.BlockSpec((_NC, _R, _D), lambda i: (0, i, 0)),
          pl.BlockSpec((_R, _D), lambda i: (i, 0)),
          pl.BlockSpec((_D, _D), lambda i: (0, 0)),
          pl.BlockSpec((1, _D), lambda i: (0, 0)),
          pl.BlockSpec((1, _D), lambda i: (0, 0)),
          pl.BlockSpec((1, _D), lambda i: (0, 0)),
      ],
      out_specs=[
          pl.BlockSpec((_R, _D), lambda i: (i, 0)),
          pl.BlockSpec((_R, _D), lambda i: (i, 0)),
      ],
      out_shape=[
          jax.ShapeDtypeStruct((_N, _D), jnp.float32),
          jax.ShapeDtypeStruct((_N, _D), jnp.float32),
      ],
  )(acc1, deg1, xr, wlT, b, g, be)


def _dense2_body(accp, invd, hr, wl, b, out):
  mean = (accp[0] + accp[1]) * invd[...]
  o = (jnp.dot(mean, wl[...], preferred_element_type=jnp.float32)
       + hr[...] + b[...])
  nrm = jnp.sqrt(jnp.sum(o * o, axis=1, keepdims=True))
  out[...] = o / jnp.maximum(nrm, 1e-12)


def _dense2(acc2, invd, hr, wlT, b):
  grid = (_N // _R,)
  return pl.pallas_call(
      lambda a, iv, hh, wl, bb, oo: _dense2_body(
          a[...], iv, hh, wl, bb, oo),
      grid=grid,
      in_specs=[
          pl.BlockSpec((_NC, _R, _D), lambda i: (0, i, 0)),
          pl.BlockSpec((_R, _D), lambda i: (i, 0)),
          pl.BlockSpec((_R, _D), lambda i: (i, 0)),
          pl.BlockSpec((_D, _D), lambda i: (0, 0)),
          pl.BlockSpec((1, _D), lambda i: (0, 0)),
      ],
      out_specs=pl.BlockSpec((_R, _D), lambda i: (i, 0)),
      out_shape=jax.ShapeDtypeStruct((_N, _D), jnp.float32),
  )(acc2, invd, hr, wlT, b)


def kernel(x, edge_index, W1l, b1l, W1r, g1, be1, W2l, b2l, W2r):
  padi = jnp.arange(_EPAD, dtype=jnp.int32)
  src2d = jnp.concatenate(
      [edge_index[0], _N + (padi % 8)]).reshape(_NBLKP, _BLK)
  dst2d = jnp.concatenate([edge_index[1], padi]).reshape(_NBLKP, _BLK)
  zrow = jnp.zeros((8, _D), jnp.float32)
  xp = jnp.concatenate([x, zrow])
  acc1 = _get_segsum()(xp, src2d, dst2d)
  deg1 = _get_deg()(dst2d)
  xr = _matmul(x, W1r.T)
  h, invd = _dense1(acc1, deg1, xr, W1l.T,
                    b1l.reshape(1, _D), g1.reshape(1, _D),
                    be1.reshape(1, _D))
  acc2 = _get_segsum()(jnp.concatenate([h, zrow]), src2d, dst2d)
  hr = _matmul(h, W2r.T)
  return _dense2(acc2, invd, hr, W2l.T, b2l.reshape(1, _D))
